# baseline (speedup 1.0000x reference)
import jax
import jax.numpy as jnp
from jax.experimental import pallas as pl

N_TOKENS = 32768
D_MODEL = 768
N_EXPERTS = 8
TEMPERATURE = 1.0

BLK = 1024
NSPLIT = 4


def _router_block(*refs):
    x_refs = refs[:NSPLIT]
    wt_ref = refs[NSPLIT]
    out_ref = refs[NSPLIT + 1]
    wt = wt_ref[...]
    for j in range(NSPLIT):
        xb = x_refs[j][...].astype(jnp.bfloat16)
        out_ref[j * BLK : (j + 1) * BLK, :] = jnp.dot(
            xb, wt, preferred_element_type=jnp.float32
        )


def kernel(x, W):
    n_tokens, d_model = x.shape
    n_experts = W.shape[0]
    wt = W.T.astype(jnp.bfloat16)

    step = BLK * NSPLIT
    grid = (n_tokens // step,)

    def x_spec(j):
        return pl.BlockSpec((BLK, d_model), lambda i, j=j: (i * NSPLIT + j, 0))

    logits = pl.pallas_call(
        _router_block,
        grid=grid,
        in_specs=[x_spec(j) for j in range(NSPLIT)]
        + [pl.BlockSpec((d_model, n_experts), lambda i: (0, 0))],
        out_specs=pl.BlockSpec((step, n_experts), lambda i: (i, 0)),
        out_shape=jax.ShapeDtypeStruct((n_tokens, n_experts), jnp.float32),
    )(*([x] * NSPLIT + [wt]))

    temp = max(TEMPERATURE, 1e-06)
    if temp != 1.0:
        logits = logits / temp
    return logits
